# baseline (speedup 1.0000x reference)
import functools

import jax
import jax.numpy as jnp
from jax import lax
from jax.experimental import pallas as pl
from jax.experimental.pallas import tpu as pltpu
from jax.experimental.pallas import tpu_sc as plsc

B, C, H, W, K = 32, 4, 256, 256, 500
HW = H * W
KP = 512
NJ = KP * C
NCHUNK = NJ // 16
NSLAB = B * C
BLK_B = 8

_NC = 2
_NS = 16
NW = _NC * _NS


def _tc_prep(f1, f2, ind, mask, target):
    half = BLK_B * C * H * 128

    def kd(f1_ref, f2_ref, fsum_ref):
        s = f1_ref[...] + f2_ref[...]
        fsum_ref[pl.ds(0, half)] = s[:, :, :, :128].reshape(half)
        fsum_ref[pl.ds(half, half)] = s[:, :, :, 128:].reshape(half)

    fsum = pl.pallas_call(
        kd,
        grid=(B // BLK_B,),
        in_specs=[
            pl.BlockSpec((BLK_B, C, H, W), lambda i: (i, 0, 0, 0)),
            pl.BlockSpec((BLK_B, C, H, W), lambda i: (i, 0, 0, 0)),
        ],
        out_specs=pl.BlockSpec((2 * half,), lambda i: (i,)),
        out_shape=jax.ShapeDtypeStruct((NSLAB * HW,), jnp.float32),
    )(f1, f2)

    def ks(ind_ref, mask_ref, tgt_ref, idx_ref, mf_ref, tgtf_ref):
        bio = lax.broadcasted_iota(jnp.int32, (B, K), 0)
        indv = ind_ref[...]
        plane = H * 128
        pos = ((bio // BLK_B) * (BLK_B * C * HW)
               + lax.bitwise_and(lax.shift_right_logical(indv, 7), 1)
               * (BLK_B * C * plane)
               + (bio % BLK_B) * (C * plane)
               + lax.shift_right_logical(indv, 8) * 128
               + lax.bitwise_and(indv, 127))
        for c in range(C):
            idx_ref[:, c * KP:c * KP + K] = pos + c * plane
            idx_ref[:, c * KP + K:(c + 1) * KP] = jnp.zeros(
                (B, KP - K), jnp.int32)
            tgtf_ref[:, c * KP:c * KP + K] = tgt_ref[:, :, c]
            tgtf_ref[:, c * KP + K:(c + 1) * KP] = jnp.zeros(
                (B, KP - K), jnp.float32)
        mf_ref[:, :K] = mask_ref[...].astype(jnp.float32)
        mf_ref[:, K:] = jnp.zeros((B, KP - K), jnp.float32)

    idx_all, mask_f, tgt_flat = pl.pallas_call(
        ks,
        out_shape=[
            jax.ShapeDtypeStruct((B, NJ), jnp.int32),
            jax.ShapeDtypeStruct((B, KP), jnp.float32),
            jax.ShapeDtypeStruct((B, NJ), jnp.float32),
        ],
    )(ind, mask, target)

    return fsum, idx_all, mask_f, tgt_flat


def _sc_partials(fsum, idx_all, mask_f, tgt):
    mesh = plsc.VectorSubcoreMesh(core_axis_name="c", subcore_axis_name="s")

    @functools.partial(
        pl.kernel,
        mesh=mesh,
        out_type=[
            jax.ShapeDtypeStruct((NW, 16), jnp.float32),
            jax.ShapeDtypeStruct((NW, 16), jnp.float32),
        ],
        scratch_types=[
            pltpu.VMEM((NJ,), jnp.int32),
            pltpu.VMEM((KP,), jnp.float32),
            pltpu.VMEM((NJ,), jnp.float32),
            pltpu.VMEM((NJ,), jnp.float32),
            pltpu.VMEM((16,), jnp.float32),
            pltpu.VMEM((16,), jnp.float32),
            pltpu.SemaphoreType.DMA,
            pltpu.SemaphoreType.DMA,
            pltpu.SemaphoreType.DMA,
            pltpu.SemaphoreType.DMA,
        ],
    )
    def k(f_hbm, idx_hbm, mask_hbm, tgt_hbm, acc_out, ms_out,
          idx_v, mask_v, tgt_v, p_v, accv, msv,
          semi, semm, semt, semg):
        wid = lax.axis_index("s") * _NC + lax.axis_index("c")
        b = wid

        cpi = pltpu.async_copy(idx_hbm.at[b], idx_v, semi)
        cpm = pltpu.async_copy(mask_hbm.at[b], mask_v, semm)
        cpt = pltpu.async_copy(tgt_hbm.at[b], tgt_v, semt)
        cpi.wait()
        cpg = pltpu.async_copy(f_hbm.at[idx_v], p_v, semg)
        cpm.wait()
        cpt.wait()
        cpg.wait()

        def comp(t, carry):
            acc, ms = carry
            m = mask_v[pl.ds(lax.rem(t, KP // 16) * 16, 16)]
            sl = pl.ds(t * 16, 16)
            e = p_v[sl] - tgt_v[sl]
            return acc + (m * e) * e, ms + m

        zero = jnp.zeros((16,), jnp.float32)
        acc, ms = lax.fori_loop(0, NCHUNK, comp, (zero, zero))
        accv[:] = acc
        msv[:] = ms
        pltpu.sync_copy(accv, acc_out.at[b])
        pltpu.sync_copy(msv, ms_out.at[b])

    return k(fsum, idx_all, mask_f, tgt)


def _tc_reduce(acc, ms):
    def k(acc_ref, ms_ref, out_ref):
        s1 = jnp.sum(acc_ref[...])
        s2 = jnp.sum(ms_ref[...])
        out_ref[0] = s1 / (s2 + 0.0001)

    return pl.pallas_call(
        k,
        out_shape=jax.ShapeDtypeStruct((1,), jnp.float32),
        out_specs=pl.BlockSpec(memory_space=pltpu.SMEM),
    )(acc, ms)


def kernel(output_stage_one, output_stage_two, mask, ind, target):
    fsum, idx_all, mask_f, tgt_flat = _tc_prep(
        output_stage_one, output_stage_two,
        ind.astype(jnp.int32), mask, target)
    acc, ms = _sc_partials(fsum, idx_all, mask_f, tgt_flat)
    return _tc_reduce(acc, ms)[0]


# speedup vs baseline: 1.0306x; 1.0306x over previous
import functools

import jax
import jax.numpy as jnp
from jax import lax
from jax.experimental import pallas as pl
from jax.experimental.pallas import tpu as pltpu
from jax.experimental.pallas import tpu_sc as plsc

B, C, H, W, K = 32, 4, 256, 256, 500
HW = H * W
KP = 512
NJ = KP * C
NCHUNK = NJ // 16
NSLAB = B * C
BLK_B = 8

_NC = 2
_NS = 16
NW = _NC * _NS


def _tc_prep(f1, f2, ind, mask):
    half = BLK_B * C * H * 128

    def kd(f1_ref, f2_ref, fsum_ref):
        s = f1_ref[...] + f2_ref[...]
        fsum_ref[pl.ds(0, half)] = s[:, :, :, :128].reshape(half)
        fsum_ref[pl.ds(half, half)] = s[:, :, :, 128:].reshape(half)

    fsum = pl.pallas_call(
        kd,
        grid=(B // BLK_B,),
        in_specs=[
            pl.BlockSpec((BLK_B, C, H, W), lambda i: (i, 0, 0, 0)),
            pl.BlockSpec((BLK_B, C, H, W), lambda i: (i, 0, 0, 0)),
        ],
        out_specs=pl.BlockSpec((2 * half,), lambda i: (i,)),
        out_shape=jax.ShapeDtypeStruct((NSLAB * HW,), jnp.float32),
    )(f1, f2)

    def ks(ind_ref, mask_ref, idx_ref, idxt_ref, mf_ref):
        kio = lax.broadcasted_iota(jnp.int32, (B, KP), 1)
        bio = lax.broadcasted_iota(jnp.int32, (B, KP), 0)
        valid = kio < K
        indv = jnp.where(
            valid, jnp.pad(ind_ref[...], ((0, 0), (0, KP - K))), 0)
        plane = H * 128
        pos = ((bio // BLK_B) * (BLK_B * C * HW)
               + lax.bitwise_and(lax.shift_right_logical(indv, 7), 1)
               * (BLK_B * C * plane)
               + (bio % BLK_B) * (C * plane)
               + lax.shift_right_logical(indv, 8) * 128
               + lax.bitwise_and(indv, 127))
        post = jnp.where(valid, bio * (K * C) + kio * C, 0)
        for c in range(C):
            idx_ref[:, c * KP:(c + 1) * KP] = pos + c * plane
            idxt_ref[:, c * KP:(c + 1) * KP] = post + c
        mf_ref[...] = jnp.where(
            valid, jnp.pad(mask_ref[...], ((0, 0), (0, KP - K))), 0
        ).astype(jnp.float32)

    idx_all, idxt_all, mask_f = pl.pallas_call(
        ks,
        out_shape=[
            jax.ShapeDtypeStruct((B, NJ), jnp.int32),
            jax.ShapeDtypeStruct((B, NJ), jnp.int32),
            jax.ShapeDtypeStruct((B, KP), jnp.float32),
        ],
    )(ind, mask)

    return fsum, idx_all, idxt_all, mask_f


def _sc_partials(fsum, tflat, idx_all, idxt_all, mask_f):
    mesh = plsc.VectorSubcoreMesh(core_axis_name="c", subcore_axis_name="s")

    @functools.partial(
        pl.kernel,
        mesh=mesh,
        out_type=[
            jax.ShapeDtypeStruct((NW, 16), jnp.float32),
            jax.ShapeDtypeStruct((NW, 16), jnp.float32),
        ],
        scratch_types=[
            pltpu.VMEM((NJ,), jnp.int32),
            pltpu.VMEM((NJ,), jnp.int32),
            pltpu.VMEM((KP,), jnp.float32),
            pltpu.VMEM((NJ,), jnp.float32),
            pltpu.VMEM((NJ,), jnp.float32),
            pltpu.VMEM((16,), jnp.float32),
            pltpu.VMEM((16,), jnp.float32),
            pltpu.SemaphoreType.DMA,
            pltpu.SemaphoreType.DMA,
            pltpu.SemaphoreType.DMA,
            pltpu.SemaphoreType.DMA,
            pltpu.SemaphoreType.DMA,
        ],
    )
    def k(f_hbm, t_hbm, idx_hbm, idxt_hbm, mask_hbm, acc_out, ms_out,
          idx_v, idxt_v, mask_v, tgt_v, p_v, accv, msv,
          semi, semit, semm, semt, semg):
        wid = lax.axis_index("s") * _NC + lax.axis_index("c")
        b = wid

        cpi = pltpu.async_copy(idx_hbm.at[b], idx_v, semi)
        cpit = pltpu.async_copy(idxt_hbm.at[b], idxt_v, semit)
        cpm = pltpu.async_copy(mask_hbm.at[b], mask_v, semm)
        cpi.wait()
        cpg = pltpu.async_copy(f_hbm.at[idx_v], p_v, semg)
        cpit.wait()
        cpt = pltpu.async_copy(t_hbm.at[idxt_v], tgt_v, semt)
        cpm.wait()
        cpt.wait()
        cpg.wait()

        def comp(t, carry):
            acc, ms = carry
            m = mask_v[pl.ds(lax.rem(t, KP // 16) * 16, 16)]
            sl = pl.ds(t * 16, 16)
            e = p_v[sl] - tgt_v[sl]
            return acc + (m * e) * e, ms + m

        zero = jnp.zeros((16,), jnp.float32)
        acc, ms = lax.fori_loop(0, NCHUNK, comp, (zero, zero))
        accv[:] = acc
        msv[:] = ms
        pltpu.sync_copy(accv, acc_out.at[b])
        pltpu.sync_copy(msv, ms_out.at[b])

    return k(fsum, tflat, idx_all, idxt_all, mask_f)


def _tc_reduce(acc, ms):
    def k(acc_ref, ms_ref, out_ref):
        s1 = jnp.sum(acc_ref[...])
        s2 = jnp.sum(ms_ref[...])
        out_ref[0] = s1 / (s2 + 0.0001)

    return pl.pallas_call(
        k,
        out_shape=jax.ShapeDtypeStruct((1,), jnp.float32),
        out_specs=pl.BlockSpec(memory_space=pltpu.SMEM),
    )(acc, ms)


def kernel(output_stage_one, output_stage_two, mask, ind, target):
    fsum, idx_all, idxt_all, mask_f = _tc_prep(
        output_stage_one, output_stage_two,
        ind.astype(jnp.int32), mask)
    tflat = target.reshape(-1)
    acc, ms = _sc_partials(fsum, tflat, idx_all, idxt_all, mask_f)
    return _tc_reduce(acc, ms)[0]
